# baseline (speedup 1.0000x reference)
import functools

import jax
import jax.numpy as jnp
from jax import lax
from jax.experimental import pallas as pl
from jax.experimental.pallas import tpu as pltpu
from jax.experimental.pallas import tpu_sc as plsc

B = 32
N = 20000
CIN = 10
COUT = 9
SCALE = 1.0 / 640.0

NC = 2
NS = 16
NW = NC * NS
BPW = (B * N) // NW
CHUNK = 2000
GROUPS = CHUNK // 16

_mesh = plsc.VectorSubcoreMesh(core_axis_name="c", subcore_axis_name="s")


@functools.partial(
    pl.kernel,
    mesh=_mesh,
    out_type=jax.ShapeDtypeStruct((B * N * COUT,), jnp.float32),
    compiler_params=pltpu.CompilerParams(needs_layout_passes=False),
    scratch_types=[
        pltpu.VMEM((CHUNK * CIN,), jnp.float32),
        pltpu.VMEM((CHUNK * COUT,), jnp.float32),
    ],
)
def _adapter(pred_hbm, out_hbm, inb, outb):
    cid = lax.axis_index("c")
    sid = lax.axis_index("s")
    wid = sid * NC + cid
    in_base = wid * (BPW * CIN)
    out_base = wid * (BPW * COUT)

    iota = lax.iota(jnp.int32, 16)
    idx_in = iota * CIN
    idx_out = iota * COUT

    def chunk_body(k, carry):
        pltpu.sync_copy(
            pred_hbm.at[pl.ds(in_base + k * (CHUNK * CIN), CHUNK * CIN)], inb)

        def group_body(g, c2):
            bi = idx_in + g * (16 * CIN)
            bo = idx_out + g * (16 * COUT)
            x1 = plsc.load_gather(inb, [bi])
            y1 = plsc.load_gather(inb, [bi + 1])
            x2 = plsc.load_gather(inb, [bi + 2])
            y2 = plsc.load_gather(inb, [bi + 3])
            lab = plsc.load_gather(inb, [bi + 4])
            a0 = plsc.load_gather(inb, [bi + 6])
            a1 = plsc.load_gather(inb, [bi + 7])
            a2 = plsc.load_gather(inb, [bi + 8])
            a3 = plsc.load_gather(inb, [bi + 9])
            plsc.store_scatter(outb, [bo], x1 * SCALE)
            plsc.store_scatter(outb, [bo + 1], y1 * SCALE)
            plsc.store_scatter(outb, [bo + 2], (x2 - x1) * SCALE)
            plsc.store_scatter(outb, [bo + 3], (y2 - y1) * SCALE)
            plsc.store_scatter(outb, [bo + 4], a0)
            plsc.store_scatter(outb, [bo + 5], a1)
            plsc.store_scatter(outb, [bo + 6], a2)
            plsc.store_scatter(outb, [bo + 7], a3)
            plsc.store_scatter(outb, [bo + 8], lab)
            return c2

        lax.fori_loop(0, GROUPS, group_body, 0)
        pltpu.sync_copy(
            outb, out_hbm.at[pl.ds(out_base + k * (CHUNK * COUT), CHUNK * COUT)])
        return carry

    lax.fori_loop(0, BPW // CHUNK, chunk_body, 0)


def kernel(predictions):
    flat = predictions.reshape(-1)
    out = _adapter(flat)
    return out.reshape(B, N, COUT)
